# baseline (speedup 1.0000x reference)
import functools

import jax
import jax.numpy as jnp
from jax import lax
from jax.experimental import pallas as pl
from jax.experimental.pallas import tpu as pltpu
from jax.experimental.pallas import tpu_sc as plsc

B = 4
SEQ = 8192
D_MODEL = 4096
HIDDEN = 4096
NUM_LAYERS = 16

SC_NC = 2
SC_NS = 16
SC_NW = SC_NC * SC_NS
SC_RPB = 512
SC_RW = B * SC_RPB // SC_NW
SC_CHUNKS = SC_RW // 16

TC_ROWS = SEQ - SC_RPB
NSTREAM = 8
R_BLK = 64
ROWS_PER_STREAM = B * TC_ROWS // NSTREAM
NS = ROWS_PER_STREAM // R_BLK

H_BLK = 512
NSUB = 4
SUB = H_BLK // NSUB
NH = HIDDEN // H_BLK
GRID = NS + NH


@functools.partial(
    pl.kernel,
    mesh=plsc.VectorSubcoreMesh(core_axis_name="c", subcore_axis_name="s"),
    out_type=jax.ShapeDtypeStruct((SC_NW, D_MODEL), jnp.float32),
    scratch_types=[
        pltpu.VMEM((16,), jnp.int32),
        pltpu.VMEM((16, D_MODEL), jnp.float32),
        pltpu.VMEM((1, D_MODEL), jnp.float32),
        pltpu.SemaphoreType.DMA,
    ],
)
def _sc_pool(x_hbm, out_hbm, idx_v, rows_v, acc_v, sem):
    c = lax.axis_index("c")
    s = lax.axis_index("s")
    wid = s * SC_NC + c
    b = wid // (SC_NW // B)
    slot = wid % (SC_NW // B)
    base = b * SEQ + (SEQ - SC_RPB) + slot * SC_RW
    lane = lax.broadcasted_iota(jnp.int32, (16,), 0)

    def chunk(g, _):
        idx_v[...] = (base + g * 16) + lane
        pltpu.async_copy(x_hbm.at[idx_v], rows_v, sem).wait()

        def col(t, _):
            vacc = acc_v[0, pl.ds(t * 16, 16)]
            for r in range(16):
                vacc = vacc + rows_v[r, pl.ds(t * 16, 16)]
            acc_v[0, pl.ds(t * 16, 16)] = vacc
            return 0

        return lax.fori_loop(0, D_MODEL // 16, col, 0)

    def zero(t, _):
        acc_v[0, pl.ds(t * 16, 16)] = jnp.zeros((16,), jnp.float32)
        return 0

    lax.fori_loop(0, D_MODEL // 16, zero, 0)
    lax.fori_loop(0, SC_CHUNKS, chunk, 0)
    pltpu.sync_copy(acc_v, out_hbm.at[pl.ds(slot * B + b, 1)])


def _router_kernel(*refs):
    x_refs = refs[:NSTREAM]
    (sc_ref,
     w1a_ref, w1b_ref, w1c_ref, w1d_ref,
     w2a_ref, w2b_ref, w2c_ref, w2d_ref,
     b1_ref, b2_ref, w3_ref, b3_ref,
     logits_ref, idx_ref, acc8_ref, xp_ref, h2_ref) = refs[NSTREAM:]
    w1_refs = (w1a_ref, w1b_ref, w1c_ref, w1d_ref)
    w2_refs = (w2a_ref, w2b_ref, w2c_ref, w2d_ref)
    i = pl.program_id(0)

    @pl.when(i < NS)
    def _pool():
        sums = [jnp.sum(x_refs[k][...], axis=0, keepdims=True)
                for k in range(NSTREAM)]

        @pl.when(i == 0)
        def _init():
            for k in range(NSTREAM):
                acc8_ref[k:k + 1, :] = sums[k]

        @pl.when(i > 0)
        def _acc():
            for k in range(NSTREAM):
                acc8_ref[k:k + 1, :] += sums[k]

    @pl.when(i >= NS)
    def _mlp():
        j = i - NS

        @pl.when(j == 0)
        def _prep():
            a = acc8_ref[...]
            tot = a[0:B, :]
            for g in range(1, NSTREAM // B):
                tot = tot + a[g * B:(g + 1) * B, :]
            sc = sc_ref[...]
            for g in range(SC_NW // B):
                tot = tot + sc[g * B:(g + 1) * B, :]
            xp_ref[...] = tot * (1.0 / SEQ)

        xp = xp_ref[...]
        part = None
        for k in range(NSUB):
            pre1 = lax.dot_general(xp, w1_refs[k][...],
                                   (((1,), (1,)), ((), ())),
                                   preferred_element_type=jnp.float32)
            h1 = jax.nn.gelu(pre1 + b1_ref[0, :, k * SUB:(k + 1) * SUB])
            p = lax.dot_general(h1, w2_refs[k][...],
                                (((1,), (1,)), ((), ())),
                                preferred_element_type=jnp.float32)
            part = p if part is None else part + p

        @pl.when(j == 0)
        def _set():
            h2_ref[...] = part

        @pl.when(j > 0)
        def _add():
            h2_ref[...] += part

        @pl.when(j == NH - 1)
        def _final():
            h2 = jax.nn.gelu(h2_ref[...] + b2_ref[...])
            logits = lax.dot_general(h2, w3_ref[...],
                                     (((1,), (1,)), ((), ())),
                                     preferred_element_type=jnp.float32)
            logits = logits + b3_ref[...]
            logits_ref[...] = logits
            col = lax.broadcasted_iota(jnp.int32, (B, NUM_LAYERS), 1)
            maxv = jnp.max(logits, axis=1, keepdims=True)
            idx_ref[...] = jnp.min(
                jnp.where(logits == maxv, col, NUM_LAYERS),
                axis=1, keepdims=True)


def _x_spec(k):
    base = ((k % B) * SEQ + (k // B) * ROWS_PER_STREAM) // R_BLK
    return pl.BlockSpec(
        (R_BLK, D_MODEL),
        lambda i, b=base: (b + jnp.minimum(i, NS - 1), 0))


def _w1_spec(k):
    return pl.BlockSpec(
        (SUB, D_MODEL),
        lambda i, k=k: (NSUB * jnp.clip(i - NS, 0, NH - 1) + k, 0))


def _w2_spec(k):
    return pl.BlockSpec(
        (HIDDEN, SUB),
        lambda i, k=k: (0, NSUB * jnp.clip(i - NS, 0, NH - 1) + k))


def kernel(x, W1, b1, W2, b2, W3, b3):
    x2 = x.reshape(B * SEQ, D_MODEL)
    b1r = b1.reshape(NH, 1, H_BLK)
    b2r = b2.reshape(1, HIDDEN)
    b3r = b3.reshape(1, NUM_LAYERS)

    sc_part = _sc_pool(x2)

    logits, idx = pl.pallas_call(
        _router_kernel,
        grid=(GRID,),
        in_specs=(
            [_x_spec(k) for k in range(NSTREAM)]
            + [pl.BlockSpec((SC_NW, D_MODEL), lambda i: (0, 0))]
            + [_w1_spec(k) for k in range(NSUB)]
            + [_w2_spec(k) for k in range(NSUB)]
            + [pl.BlockSpec((1, 1, H_BLK),
                            lambda i: (jnp.clip(i - NS, 0, NH - 1), 0, 0)),
               pl.BlockSpec((1, HIDDEN), lambda i: (0, 0)),
               pl.BlockSpec((NUM_LAYERS, HIDDEN), lambda i: (0, 0)),
               pl.BlockSpec((1, NUM_LAYERS), lambda i: (0, 0))]
        ),
        out_specs=[
            pl.BlockSpec((B, NUM_LAYERS), lambda i: (0, 0)),
            pl.BlockSpec((B, 1), lambda i: (0, 0)),
        ],
        out_shape=[
            jax.ShapeDtypeStruct((B, NUM_LAYERS), jnp.float32),
            jax.ShapeDtypeStruct((B, 1), jnp.int32),
        ],
        scratch_shapes=[
            pltpu.VMEM((NSTREAM, D_MODEL), jnp.float32),
            pltpu.VMEM((B, D_MODEL), jnp.float32),
            pltpu.VMEM((B, HIDDEN), jnp.float32),
        ],
        compiler_params=pltpu.CompilerParams(
            dimension_semantics=("arbitrary",)),
    )(*([x2] * NSTREAM), sc_part,
      W1, W1, W1, W1, W2, W2, W2, W2, b1r, b2r, W3, b3r)

    return (idx.reshape(B), logits)


# speedup vs baseline: 1.1182x; 1.1182x over previous
import functools

import jax
import jax.numpy as jnp
from jax import lax
from jax.experimental import pallas as pl
from jax.experimental.pallas import tpu as pltpu
from jax.experimental.pallas import tpu_sc as plsc

B = 4
SEQ = 8192
D_MODEL = 4096
HIDDEN = 4096
NUM_LAYERS = 16

SC_NC = 2
SC_NS = 16
SC_NW = SC_NC * SC_NS
SC_RPB = 512
SC_RW = B * SC_RPB // SC_NW
SC_CHUNKS = SC_RW // 16

TC_ROWS = SEQ - SC_RPB
NSTREAM = 8
R_BLK = 64
ROWS_PER_STREAM = B * TC_ROWS // NSTREAM
NS = ROWS_PER_STREAM // R_BLK

H_BLK = 512
NSUB = 4
SUB = H_BLK // NSUB
NH = HIDDEN // H_BLK
GRID = NS + NH


@functools.partial(
    pl.kernel,
    mesh=plsc.VectorSubcoreMesh(core_axis_name="c", subcore_axis_name="s"),
    out_type=jax.ShapeDtypeStruct((SC_NW, D_MODEL), jnp.float32),
    scratch_types=[
        pltpu.VMEM((16,), jnp.int32),
        pltpu.VMEM((16, D_MODEL), jnp.float32),
        pltpu.VMEM((1, D_MODEL), jnp.float32),
        pltpu.SemaphoreType.DMA,
    ],
)
def _sc_pool(x_hbm, out_hbm, idx_v, rows_v, acc_v, sem):
    c = lax.axis_index("c")
    s = lax.axis_index("s")
    wid = s * SC_NC + c
    b = wid // (SC_NW // B)
    slot = wid % (SC_NW // B)
    base = b * SEQ + (SEQ - SC_RPB) + slot * SC_RW
    lane = lax.broadcasted_iota(jnp.int32, (16,), 0)

    def chunk(g, _):
        idx_v[...] = (base + g * 16) + lane
        pltpu.async_copy(x_hbm.at[idx_v], rows_v, sem).wait()

        def col(t, _):
            vacc = acc_v[0, pl.ds(t * 16, 16)]
            for r in range(16):
                vacc = vacc + rows_v[r, pl.ds(t * 16, 16)]
            acc_v[0, pl.ds(t * 16, 16)] = vacc
            return 0

        return lax.fori_loop(0, D_MODEL // 16, col, 0)

    def zero(t, _):
        acc_v[0, pl.ds(t * 16, 16)] = jnp.zeros((16,), jnp.float32)
        return 0

    lax.fori_loop(0, D_MODEL // 16, zero, 0)
    lax.fori_loop(0, SC_CHUNKS, chunk, 0)
    pltpu.sync_copy(acc_v, out_hbm.at[pl.ds(slot * B + b, 1)])


def _pool_kernel(*refs):
    x_refs, acc8_ref = refs[:NSTREAM], refs[NSTREAM]
    i = pl.program_id(0)
    sums = [jnp.sum(x_refs[k][...], axis=0, keepdims=True)
            for k in range(NSTREAM)]

    @pl.when(i == 0)
    def _init():
        for k in range(NSTREAM):
            acc8_ref[k:k + 1, :] = sums[k]

    @pl.when(i > 0)
    def _acc():
        for k in range(NSTREAM):
            acc8_ref[k:k + 1, :] += sums[k]


def _mlp_kernel(acc8_ref, sc_ref,
                w1a_ref, w1b_ref, w1c_ref, w1d_ref,
                w2a_ref, w2b_ref, w2c_ref, w2d_ref,
                b1_ref, b2_ref, w3_ref, b3_ref,
                logits_ref, idx_ref, xp_ref, h2_ref):
    w1_refs = (w1a_ref, w1b_ref, w1c_ref, w1d_ref)
    w2_refs = (w2a_ref, w2b_ref, w2c_ref, w2d_ref)
    j = pl.program_id(0)

    if True:
        @pl.when(j == 0)
        def _prep():
            a = acc8_ref[...]
            tot = a[0:B, :]
            for g in range(1, NSTREAM // B):
                tot = tot + a[g * B:(g + 1) * B, :]
            sc = sc_ref[...]
            for g in range(SC_NW // B):
                tot = tot + sc[g * B:(g + 1) * B, :]
            xp_ref[...] = tot * (1.0 / SEQ)

        xp = xp_ref[...]
        part = None
        for k in range(NSUB):
            pre1 = lax.dot_general(xp, w1_refs[k][...],
                                   (((1,), (1,)), ((), ())),
                                   preferred_element_type=jnp.float32)
            h1 = jax.nn.gelu(pre1 + b1_ref[0, :, k * SUB:(k + 1) * SUB])
            p = lax.dot_general(h1, w2_refs[k][...],
                                (((1,), (1,)), ((), ())),
                                preferred_element_type=jnp.float32)
            part = p if part is None else part + p

        @pl.when(j == 0)
        def _set():
            h2_ref[...] = part

        @pl.when(j > 0)
        def _add():
            h2_ref[...] += part

        @pl.when(j == NH - 1)
        def _final():
            h2 = jax.nn.gelu(h2_ref[...] + b2_ref[...])
            logits = lax.dot_general(h2, w3_ref[...],
                                     (((1,), (1,)), ((), ())),
                                     preferred_element_type=jnp.float32)
            logits = logits + b3_ref[...]
            logits_ref[...] = logits
            col = lax.broadcasted_iota(jnp.int32, (B, NUM_LAYERS), 1)
            maxv = jnp.max(logits, axis=1, keepdims=True)
            idx_ref[...] = jnp.min(
                jnp.where(logits == maxv, col, NUM_LAYERS),
                axis=1, keepdims=True)


def _x_spec(k):
    base = ((k % B) * SEQ + (k // B) * ROWS_PER_STREAM) // R_BLK
    return pl.BlockSpec(
        (R_BLK, D_MODEL),
        lambda i, b=base: (b + jnp.minimum(i, NS - 1), 0))


def _w1_spec(k):
    return pl.BlockSpec(
        (SUB, D_MODEL),
        lambda j, k=k: (NSUB * j + k, 0))


def _w2_spec(k):
    return pl.BlockSpec(
        (HIDDEN, SUB),
        lambda j, k=k: (0, NSUB * j + k))


def kernel(x, W1, b1, W2, b2, W3, b3):
    x2 = x.reshape(B * SEQ, D_MODEL)
    b1r = b1.reshape(NH, 1, H_BLK)
    b2r = b2.reshape(1, HIDDEN)
    b3r = b3.reshape(1, NUM_LAYERS)

    sc_part = _sc_pool(x2)

    acc8 = pl.pallas_call(
        _pool_kernel,
        grid=(NS,),
        in_specs=[_x_spec(k) for k in range(NSTREAM)],
        out_specs=pl.BlockSpec((NSTREAM, D_MODEL), lambda i: (0, 0)),
        out_shape=jax.ShapeDtypeStruct((NSTREAM, D_MODEL), jnp.float32),
        compiler_params=pltpu.CompilerParams(
            dimension_semantics=("arbitrary",)),
    )(*([x2] * NSTREAM))

    logits, idx = pl.pallas_call(
        _mlp_kernel,
        grid=(NH,),
        in_specs=(
            [pl.BlockSpec((NSTREAM, D_MODEL), lambda j: (0, 0)),
             pl.BlockSpec((SC_NW, D_MODEL), lambda j: (0, 0))]
            + [_w1_spec(k) for k in range(NSUB)]
            + [_w2_spec(k) for k in range(NSUB)]
            + [pl.BlockSpec((1, 1, H_BLK), lambda j: (j, 0, 0)),
               pl.BlockSpec((1, HIDDEN), lambda j: (0, 0)),
               pl.BlockSpec((NUM_LAYERS, HIDDEN), lambda j: (0, 0)),
               pl.BlockSpec((1, NUM_LAYERS), lambda j: (0, 0))]
        ),
        out_specs=[
            pl.BlockSpec((B, NUM_LAYERS), lambda j: (0, 0)),
            pl.BlockSpec((B, 1), lambda j: (0, 0)),
        ],
        out_shape=[
            jax.ShapeDtypeStruct((B, NUM_LAYERS), jnp.float32),
            jax.ShapeDtypeStruct((B, 1), jnp.int32),
        ],
        scratch_shapes=[
            pltpu.VMEM((B, D_MODEL), jnp.float32),
            pltpu.VMEM((B, HIDDEN), jnp.float32),
        ],
        compiler_params=pltpu.CompilerParams(
            dimension_semantics=("arbitrary",)),
    )(acc8, sc_part, W1, W1, W1, W1, W2, W2, W2, W2, b1r, b2r, W3, b3r)

    return (idx.reshape(B), logits)


# speedup vs baseline: 1.2815x; 1.1460x over previous
import jax
import jax.numpy as jnp
from jax import lax
from jax.experimental import pallas as pl
from jax.experimental.pallas import tpu as pltpu

B = 4
SEQ = 8192
D_MODEL = 4096
HIDDEN = 4096
NUM_LAYERS = 16

NSTREAM = 8
R_BLK = 64
ROWS_PER_STREAM = B * SEQ // NSTREAM
NS = ROWS_PER_STREAM // R_BLK

H_BLK = 512
NSUB = 4
SUB = H_BLK // NSUB
NH = HIDDEN // H_BLK
GRID = NS + NH


def _router_kernel(*refs):
    x_refs = refs[:NSTREAM]
    (w1a_ref, w1b_ref, w1c_ref, w1d_ref,
     w2a_ref, w2b_ref, w2c_ref, w2d_ref,
     b1_ref, b2_ref, w3_ref, b3_ref,
     logits_ref, idx_ref, acc8_ref, xp_ref, h2_ref) = refs[NSTREAM:]
    w1_refs = (w1a_ref, w1b_ref, w1c_ref, w1d_ref)
    w2_refs = (w2a_ref, w2b_ref, w2c_ref, w2d_ref)
    i = pl.program_id(0)

    @pl.when(i < NS)
    def _pool():
        sums = [jnp.sum(x_refs[k][...], axis=0, keepdims=True)
                for k in range(NSTREAM)]

        @pl.when(i == 0)
        def _init():
            for k in range(NSTREAM):
                acc8_ref[k:k + 1, :] = sums[k]

        @pl.when(i > 0)
        def _acc():
            for k in range(NSTREAM):
                acc8_ref[k:k + 1, :] += sums[k]

    @pl.when(i >= NS)
    def _mlp():
        j = i - NS

        @pl.when(j == 0)
        def _prep():
            a = acc8_ref[...]
            tot = a[0:B, :]
            for g in range(1, NSTREAM // B):
                tot = tot + a[g * B:(g + 1) * B, :]
            xp_ref[...] = tot * (1.0 / SEQ)

        xp = xp_ref[...]
        part = None
        for k in range(NSUB):
            pre1 = lax.dot_general(xp, w1_refs[k][...],
                                   (((1,), (1,)), ((), ())),
                                   preferred_element_type=jnp.float32)
            h1 = jax.nn.gelu(pre1 + b1_ref[0, :, k * SUB:(k + 1) * SUB])
            p = lax.dot_general(h1, w2_refs[k][...],
                                (((1,), (1,)), ((), ())),
                                preferred_element_type=jnp.float32)
            part = p if part is None else part + p

        @pl.when(j == 0)
        def _set():
            h2_ref[...] = part

        @pl.when(j > 0)
        def _add():
            h2_ref[...] += part

        @pl.when(j == NH - 1)
        def _final():
            h2 = jax.nn.gelu(h2_ref[...] + b2_ref[...])
            logits = lax.dot_general(h2, w3_ref[...],
                                     (((1,), (1,)), ((), ())),
                                     preferred_element_type=jnp.float32)
            logits = logits + b3_ref[...]
            logits_ref[...] = logits
            col = lax.broadcasted_iota(jnp.int32, (B, NUM_LAYERS), 1)
            maxv = jnp.max(logits, axis=1, keepdims=True)
            idx_ref[...] = jnp.min(
                jnp.where(logits == maxv, col, NUM_LAYERS),
                axis=1, keepdims=True)


def _x_spec(k):
    base = ((k % B) * SEQ + (k // B) * ROWS_PER_STREAM) // R_BLK
    return pl.BlockSpec(
        (R_BLK, D_MODEL),
        lambda i, b=base: (b + jnp.minimum(i, NS - 1), 0))


def _w1_spec(k):
    return pl.BlockSpec(
        (SUB, D_MODEL),
        lambda i, k=k: (NSUB * jnp.clip(i - NS, 0, NH - 1) + k, 0))


def _w2_spec(k):
    return pl.BlockSpec(
        (HIDDEN, SUB),
        lambda i, k=k: (0, NSUB * jnp.clip(i - NS, 0, NH - 1) + k))


def kernel(x, W1, b1, W2, b2, W3, b3):
    x2 = x.reshape(B * SEQ, D_MODEL)
    b1r = b1.reshape(NH, 1, H_BLK)
    b2r = b2.reshape(1, HIDDEN)
    b3r = b3.reshape(1, NUM_LAYERS)

    logits, idx = pl.pallas_call(
        _router_kernel,
        grid=(GRID,),
        in_specs=(
            [_x_spec(k) for k in range(NSTREAM)]
            + [_w1_spec(k) for k in range(NSUB)]
            + [_w2_spec(k) for k in range(NSUB)]
            + [pl.BlockSpec((1, 1, H_BLK),
                            lambda i: (jnp.clip(i - NS, 0, NH - 1), 0, 0)),
               pl.BlockSpec((1, HIDDEN), lambda i: (0, 0)),
               pl.BlockSpec((NUM_LAYERS, HIDDEN), lambda i: (0, 0)),
               pl.BlockSpec((1, NUM_LAYERS), lambda i: (0, 0))]
        ),
        out_specs=[
            pl.BlockSpec((B, NUM_LAYERS), lambda i: (0, 0)),
            pl.BlockSpec((B, 1), lambda i: (0, 0)),
        ],
        out_shape=[
            jax.ShapeDtypeStruct((B, NUM_LAYERS), jnp.float32),
            jax.ShapeDtypeStruct((B, 1), jnp.int32),
        ],
        scratch_shapes=[
            pltpu.VMEM((NSTREAM, D_MODEL), jnp.float32),
            pltpu.VMEM((B, D_MODEL), jnp.float32),
            pltpu.VMEM((B, HIDDEN), jnp.float32),
        ],
        compiler_params=pltpu.CompilerParams(
            dimension_semantics=("arbitrary",)),
    )(*([x2] * NSTREAM), W1, W1, W1, W1, W2, W2, W2, W2, b1r, b2r, W3, b3r)

    return (idx.reshape(B), logits)
